# baseline (speedup 1.0000x reference)
import jax
import jax.numpy as jnp
from jax.experimental import pallas as pl
from jax.experimental.pallas import tpu as pltpu

_SRC_X = 2048
_SRC_Y = 2048
_ROWS = 4096
_COLS = 4096
_CHR = 256
_NBUF = 20
_NCHUNK = _ROWS // _CHR
_SRC_CHUNK = _SRC_X // _CHR
_SRC_R = _SRC_X % _CHR


def _body(b_hbm, bt_smem, o_hbm, *scratch):
    bufs = scratch[:_NBUF]
    sin = scratch[_NBUF : 2 * _NBUF]
    sout = scratch[2 * _NBUF :]

    def in_cp(g):
        return pltpu.make_async_copy(
            b_hbm.at[:, pl.ds(g * _CHR, _CHR), :], bufs[g % _NBUF], sin[g % _NBUF]
        )

    def out_cp(g):
        return pltpu.make_async_copy(
            bufs[g % _NBUF], o_hbm.at[:, pl.ds(g * _CHR, _CHR), :], sout[g % _NBUF]
        )

    for g in range(_NBUF):
        in_cp(g).start()

    for g in range(_NCHUNK):
        in_cp(g).wait()
        if g == _SRC_CHUNK:
            buf = bufs[g % _NBUF]
            col_ids = jax.lax.broadcasted_iota(jnp.int32, (1, _COLS), 1)
            buf[0, _SRC_R : _SRC_R + 1, :] = jnp.where(
                col_ids == _SRC_Y, bt_smem[0, 0], buf[0, _SRC_R : _SRC_R + 1, :]
            )
        out_cp(g).start()
        if g + _NBUF < _NCHUNK:
            out_cp(g).wait()
            in_cp(g + _NBUF).start()

    for g in range(_NCHUNK - _NBUF, _NCHUNK):
        out_cp(g).wait()


def kernel(B, Bt):
    scratch = (
        [pltpu.VMEM((1, _CHR, _COLS), jnp.float32) for _ in range(_NBUF)]
        + [pltpu.SemaphoreType.DMA for _ in range(2 * _NBUF)]
    )
    return pl.pallas_call(
        _body,
        in_specs=[
            pl.BlockSpec(memory_space=pl.ANY),
            pl.BlockSpec(memory_space=pltpu.SMEM),
        ],
        out_specs=pl.BlockSpec(memory_space=pl.ANY),
        out_shape=jax.ShapeDtypeStruct((1, _ROWS, _COLS), jnp.float32),
        scratch_shapes=scratch,
    )(B, Bt)
